# Initial kernel scaffold; baseline (speedup 1.0000x reference)
import functools

import jax
import jax.numpy as jnp
from jax import lax
from jax.experimental import pallas as pl
from jax.experimental.pallas import tpu as pltpu
from jax.experimental.pallas import tpu_sc as plsc

N = 10000
D = 128
E = 640000

NC = 2
NS = 16
E_SC = E // NC
E_TILE = E_SC // NS
E_BLK = 80
N_CHUNK = E_TILE // E_BLK
ROWS_TILE = N // NS

_mesh = plsc.VectorSubcoreMesh(core_axis_name="c", subcore_axis_name="s")


@functools.partial(
    pl.kernel,
    out_type=jax.ShapeDtypeStruct((NC * N, D), jnp.float32),
    mesh=_mesh,
    scratch_types=[
        pltpu.VMEM_SHARED((N, D), jnp.float32),
        pltpu.VMEM((E_BLK,), jnp.int32),
        pltpu.VMEM((E_BLK,), jnp.int32),
        pltpu.VMEM((E_BLK,), jnp.float32),
        pltpu.VMEM((E_BLK, D), jnp.float32),
        pltpu.SemaphoreType.DMA,
    ],
)
def _spmm_layer(x_hbm, rows_hbm, cols_hbm, vals_hbm, zeros_hbm, out_hbm,
                acc, idx_c, idx_r, val_v, gbuf, sem):
    c = lax.axis_index("c")
    s = lax.axis_index("s")

    pltpu.sync_copy(zeros_hbm, acc.at[pl.ds(s * ROWS_TILE, ROWS_TILE)])
    plsc.subcore_barrier()

    base_e = c * E_SC + s * E_TILE

    def chunk_body(i, carry):
        off = base_e + i * E_BLK
        pltpu.sync_copy(cols_hbm.at[pl.ds(off, E_BLK)], idx_c)
        pltpu.sync_copy(rows_hbm.at[pl.ds(off, E_BLK)], idx_r)
        pltpu.sync_copy(vals_hbm.at[pl.ds(off, E_BLK)], val_v)
        pltpu.async_copy(x_hbm.at[idx_c], gbuf, sem).wait()

        def scale_body(e, carry2):
            v = plsc.load_gather(val_v, [jnp.full((16,), e, jnp.int32)])
            for d in range(D // 16):
                sl = pl.ds(d * 16, 16)
                gbuf[e, sl] = gbuf[e, sl] * v
            return carry2

        lax.fori_loop(0, E_BLK, scale_body, 0)
        pltpu.sync_copy(gbuf, acc.at[idx_r], add=True)
        return carry

    lax.fori_loop(0, N_CHUNK, chunk_body, 0)
    plsc.subcore_barrier()

    row0 = s * ROWS_TILE
    pltpu.sync_copy(acc.at[pl.ds(row0, ROWS_TILE)],
                    out_hbm.at[pl.ds(c * N + row0, ROWS_TILE)])


_BLK = 1000


def _add2_body(a_ref, b_ref, o_ref):
    o_ref[...] = a_ref[...] + b_ref[...]


def _combine_partials(p):
    return pl.pallas_call(
        _add2_body,
        out_shape=jax.ShapeDtypeStruct((N, D), jnp.float32),
        grid=(N // _BLK,),
        in_specs=[
            pl.BlockSpec((_BLK, D), lambda i: (i, 0)),
            pl.BlockSpec((_BLK, D), lambda i: (i + N // _BLK, 0)),
        ],
        out_specs=pl.BlockSpec((_BLK, D), lambda i: (i, 0)),
    )(p, p)


def _mean_body(x0_ref, x1_ref, a_ref, b_ref, o_ref):
    o_ref[...] = (x0_ref[...] + x1_ref[...] + a_ref[...] + b_ref[...]) * (1.0 / 3.0)


def _final_mean(x0, x1, p2):
    return pl.pallas_call(
        _mean_body,
        out_shape=jax.ShapeDtypeStruct((N, D), jnp.float32),
        grid=(N // _BLK,),
        in_specs=[
            pl.BlockSpec((_BLK, D), lambda i: (i, 0)),
            pl.BlockSpec((_BLK, D), lambda i: (i, 0)),
            pl.BlockSpec((_BLK, D), lambda i: (i, 0)),
            pl.BlockSpec((_BLK, D), lambda i: (i + N // _BLK, 0)),
        ],
        out_specs=pl.BlockSpec((_BLK, D), lambda i: (i, 0)),
    )(x0, x1, p2, p2)


def kernel(adj1_indices, adj1_values, adj2_indices, adj2_values, user_emb, item_emb):
    rows = jnp.concatenate([adj1_indices[0], adj2_indices[0]], axis=0)
    cols = jnp.concatenate([adj1_indices[1], adj2_indices[1]], axis=0)
    vals = jnp.concatenate([adj1_values, adj2_values], axis=0)
    x0 = jnp.concatenate([item_emb, user_emb], axis=0)
    zeros = jnp.zeros((ROWS_TILE, D), jnp.float32)

    p1 = _spmm_layer(x0, rows, cols, vals, zeros)
    x1 = _combine_partials(p1)
    p2 = _spmm_layer(x1, rows, cols, vals, zeros)
    return _final_mean(x0, x1, p2)


# speedup vs baseline: 5.0947x; 5.0947x over previous
import functools

import jax
import jax.numpy as jnp
from jax import lax
from jax.experimental import pallas as pl
from jax.experimental.pallas import tpu as pltpu
from jax.experimental.pallas import tpu_sc as plsc

N = 10000
NP = 10240
D = 128
E = 640000

NC = 2
NS = 16
E_SC = E // NC
E_TILE = E_SC // NS
E_BLK = 80
N_CHUNK = E_TILE // E_BLK
ROWS_TILE = NP // NS

_mesh = plsc.VectorSubcoreMesh(core_axis_name="c", subcore_axis_name="s")


@functools.partial(
    pl.kernel,
    out_type=jax.ShapeDtypeStruct((NC * NP, D), jnp.float32),
    mesh=_mesh,
    scratch_types=[
        pltpu.VMEM_SHARED((NP, D), jnp.float32),
        pltpu.VMEM((E_BLK,), jnp.int32),
        pltpu.VMEM((E_BLK,), jnp.int32),
        pltpu.VMEM((E_BLK,), jnp.float32),
        pltpu.VMEM((E_BLK, D), jnp.float32),
        pltpu.SemaphoreType.DMA,
    ],
)
def _spmm_layer(x_hbm, rows_hbm, cols_hbm, vals_hbm, zeros_hbm, out_hbm,
                acc, idx_c, idx_r, val_v, gbuf, sem):
    c = lax.axis_index("c")
    s = lax.axis_index("s")

    pltpu.sync_copy(zeros_hbm, acc.at[pl.ds(s * ROWS_TILE, ROWS_TILE)])
    plsc.subcore_barrier()

    base_e = c * E_SC + s * E_TILE

    def chunk_body(i, carry):
        off = base_e + i * E_BLK
        pltpu.sync_copy(cols_hbm.at[pl.ds(off, E_BLK)], idx_c)
        pltpu.sync_copy(rows_hbm.at[pl.ds(off, E_BLK)], idx_r)
        pltpu.sync_copy(vals_hbm.at[pl.ds(off, E_BLK)], val_v)
        pltpu.async_copy(x_hbm.at[idx_c], gbuf, sem).wait()

        def scale_group(g, carry2):
            v16 = val_v[pl.ds(g * 16, 16)]
            for j in range(16):
                e = g * 16 + j
                v = v16[j]
                for d in range(D // 16):
                    sl = pl.ds(d * 16, 16)
                    gbuf[e, sl] = gbuf[e, sl] * v
            return carry2

        lax.fori_loop(0, E_BLK // 16, scale_group, 0)
        pltpu.sync_copy(gbuf, acc.at[idx_r], add=True)
        return carry

    lax.fori_loop(0, N_CHUNK, chunk_body, 0)
    plsc.subcore_barrier()

    row0 = s * ROWS_TILE
    pltpu.sync_copy(acc.at[pl.ds(row0, ROWS_TILE)],
                    out_hbm.at[pl.ds(c * NP + row0, ROWS_TILE)])


_BLK = 1000
_BLKP = 1024


def _add2_body(a_ref, b_ref, o_ref):
    o_ref[...] = a_ref[...] + b_ref[...]


def _combine_partials(p):
    return pl.pallas_call(
        _add2_body,
        out_shape=jax.ShapeDtypeStruct((NP, D), jnp.float32),
        grid=(NP // _BLKP,),
        in_specs=[
            pl.BlockSpec((_BLKP, D), lambda i: (i, 0)),
            pl.BlockSpec((_BLKP, D), lambda i: (i + NP // _BLKP, 0)),
        ],
        out_specs=pl.BlockSpec((_BLKP, D), lambda i: (i, 0)),
    )(p, p)


def _mean_body(x0_ref, x1_ref, a_ref, b_ref, o_ref):
    o_ref[...] = (x0_ref[...] + x1_ref[...] + a_ref[...] + b_ref[...]) * (1.0 / 3.0)


def _final_mean(x0, x1, p2a, p2b):
    return pl.pallas_call(
        _mean_body,
        out_shape=jax.ShapeDtypeStruct((N, D), jnp.float32),
        grid=(N // _BLK,),
        in_specs=[pl.BlockSpec((_BLK, D), lambda i: (i, 0))] * 4,
        out_specs=pl.BlockSpec((_BLK, D), lambda i: (i, 0)),
    )(x0, x1, p2a, p2b)


def kernel(adj1_indices, adj1_values, adj2_indices, adj2_values, user_emb, item_emb):
    rows = jnp.concatenate([adj1_indices[0], adj2_indices[0]], axis=0)
    cols = jnp.concatenate([adj1_indices[1], adj2_indices[1]], axis=0)
    vals = jnp.concatenate([adj1_values, adj2_values], axis=0)
    x0 = jnp.concatenate([item_emb, user_emb], axis=0)
    zeros = jnp.zeros((ROWS_TILE, D), jnp.float32)

    p1 = _spmm_layer(x0, rows, cols, vals, zeros)
    x1 = _combine_partials(p1)
    p2 = _spmm_layer(x1, rows, cols, vals, zeros)
    return _final_mean(x0, x1[:N], p2[:N], p2[NP:NP + N])
